# baseline (speedup 1.0000x reference)
import functools

import jax
import jax.numpy as jnp
from jax import lax
from jax.experimental import pallas as pl
from jax.experimental.pallas import tpu as pltpu
from jax.experimental.pallas import tpu_sc as plsc

_N = 100000
_D = 256
_NC, _NS = 2, 16
_CH = 160
_NBUF = 3
_FAST_NCH = 28
_SLOW_NCH = 12
_FAST_CORE = 0
_TOT_CH = _NS * (_FAST_NCH + _SLOW_NCH)
_IDX_PAD = (_TOT_CH + 1) * _CH
_FULL = (_N // _CH) * _CH
_TAIL = _N - _FULL

_mesh = plsc.VectorSubcoreMesh(core_axis_name="c", subcore_axis_name="s")


@functools.partial(
    pl.kernel,
    mesh=_mesh,
    out_type=jax.ShapeDtypeStruct((_N, _D), jnp.float32),
    scratch_types=[
        pltpu.VMEM((_FAST_NCH * _CH,), jnp.int32),
        pltpu.VMEM((_NBUF, _CH, _D), jnp.float32),
    ] + [pltpu.SemaphoreType.DMA] * _NBUF,
)
def _emb_gather(idx_hbm, emb_hbm, out_hbm, idx_v, rows_v, *sems):
    cid = lax.axis_index("c")
    sid = lax.axis_index("s")
    on_fast = cid == _FAST_CORE
    my_nch = jnp.where(on_fast, _FAST_NCH, _SLOW_NCH)
    chunk0 = jnp.where(on_fast, sid * _FAST_NCH,
                       _NS * _FAST_NCH + sid * _SLOW_NCH)

    @pl.when(on_fast)
    def _stage_fast():
        pltpu.sync_copy(idx_hbm.at[pl.ds(chunk0 * _CH, _FAST_NCH * _CH)],
                        idx_v)

    @pl.when(jnp.logical_not(on_fast))
    def _stage_slow():
        pltpu.sync_copy(idx_hbm.at[pl.ds(chunk0 * _CH, _SLOW_NCH * _CH)],
                        idx_v.at[pl.ds(0, _SLOW_NCH * _CH)])

    def _issue(k):
        b = k % _NBUF

        @pl.when(k < my_nch)
        def _():
            pltpu.async_copy(emb_hbm.at[idx_v.at[pl.ds(k * _CH, _CH)]],
                             rows_v.at[b], sems[b])

    for k in range(_NBUF - 1):
        _issue(k)

    for j in range(_FAST_NCH):
        b = j % _NBUF

        if j + _NBUF - 1 < _FAST_NCH:
            _issue(j + _NBUF - 1)

        @pl.when(j < my_nch)
        def _wait_and_write(b=b, j=j):
            pltpu.make_async_copy(emb_hbm.at[idx_v.at[pl.ds(j * _CH, _CH)]],
                                  rows_v.at[b], sems[b]).wait()
            rbase = (chunk0 + j) * _CH

            @pl.when(rbase + _CH <= _N)
            def _full_write():
                pltpu.sync_copy(rows_v.at[b], out_hbm.at[pl.ds(rbase, _CH)])

            if _TAIL:
                @pl.when(rbase == _FULL)
                def _tail_write():
                    pltpu.sync_copy(rows_v.at[b].at[pl.ds(0, _TAIL)],
                                    out_hbm.at[pl.ds(_FULL, _TAIL)])


def kernel(node_val, emb):
    idx = node_val.reshape(-1).astype(jnp.int32)
    idx = jnp.pad(idx, (0, _IDX_PAD - _N))
    return _emb_gather(idx, emb)


# speedup vs baseline: 1.9698x; 1.9698x over previous
import functools

import jax
import jax.numpy as jnp
from jax import lax
from jax.experimental import pallas as pl
from jax.experimental.pallas import tpu as pltpu
from jax.experimental.pallas import tpu_sc as plsc

_N = 100000
_D = 256
_NC, _NS = 2, 16
_CH = 128
_NBUF = 3
_FAST_NCH = 34
_SLOW_NCH = 15
_FAST_CORE = 0
_TOT_CH = _NS * (_FAST_NCH + _SLOW_NCH)
_IDX_PAD = (_TOT_CH + 1) * _CH
_FULL = (_N // _CH) * _CH
_TAIL = _N - _FULL

_mesh = plsc.VectorSubcoreMesh(core_axis_name="c", subcore_axis_name="s")


@functools.partial(
    pl.kernel,
    mesh=_mesh,
    out_type=jax.ShapeDtypeStruct((_N, _D), jnp.float32),
    scratch_types=[
        pltpu.VMEM((_FAST_NCH * _CH,), jnp.int32),
        pltpu.VMEM((_NBUF, _CH, _D), jnp.float32),
    ] + [pltpu.SemaphoreType.DMA] * _NBUF,
)
def _emb_gather(idx_hbm, emb_hbm, out_hbm, idx_v, rows_v, *sems):
    cid = lax.axis_index("c")
    sid = lax.axis_index("s")
    on_fast = cid == _FAST_CORE
    my_nch = jnp.where(on_fast, _FAST_NCH, _SLOW_NCH)
    chunk0 = jnp.where(on_fast, sid * _FAST_NCH,
                       _NS * _FAST_NCH + sid * _SLOW_NCH)

    @pl.when(on_fast)
    def _stage_fast():
        pltpu.sync_copy(idx_hbm.at[pl.ds(chunk0 * _CH, _FAST_NCH * _CH)],
                        idx_v)

    @pl.when(jnp.logical_not(on_fast))
    def _stage_slow():
        pltpu.sync_copy(idx_hbm.at[pl.ds(chunk0 * _CH, _SLOW_NCH * _CH)],
                        idx_v.at[pl.ds(0, _SLOW_NCH * _CH)])

    def _issue(k):
        b = k % _NBUF

        @pl.when(k < my_nch)
        def _():
            pltpu.async_copy(emb_hbm.at[idx_v.at[pl.ds(k * _CH, _CH)]],
                             rows_v.at[b], sems[b])

    for k in range(_NBUF - 1):
        _issue(k)

    for j in range(_FAST_NCH):
        b = j % _NBUF

        if j + _NBUF - 1 < _FAST_NCH:
            _issue(j + _NBUF - 1)

        @pl.when(j < my_nch)
        def _wait_and_write(b=b, j=j):
            pltpu.make_async_copy(emb_hbm.at[idx_v.at[pl.ds(j * _CH, _CH)]],
                                  rows_v.at[b], sems[b]).wait()
            rbase = (chunk0 + j) * _CH

            @pl.when(rbase + _CH <= _N)
            def _full_write():
                pltpu.sync_copy(rows_v.at[b], out_hbm.at[pl.ds(rbase, _CH)])

            if _TAIL:
                @pl.when(rbase == _FULL)
                def _tail_write():
                    pltpu.sync_copy(rows_v.at[b].at[pl.ds(0, _TAIL)],
                                    out_hbm.at[pl.ds(_FULL, _TAIL)])


def kernel(node_val, emb):
    idx = node_val.reshape(-1).astype(jnp.int32)
    idx = jnp.pad(idx, (0, _IDX_PAD - _N))
    return _emb_gather(idx, emb)


# speedup vs baseline: 2.1080x; 1.0702x over previous
import functools

import jax
import jax.numpy as jnp
from jax import lax
from jax.experimental import pallas as pl
from jax.experimental.pallas import tpu as pltpu
from jax.experimental.pallas import tpu_sc as plsc

_N = 100000
_D = 256
_NC, _NS = 2, 16
_CH = 128
_NBUF = 3
_FAST_CORE = 0
_TOT_CH = -(-_N // _CH)
_FAST_CH = 551
_FAST_HI = 35
_SLOW_HI = 15
_FULL = (_N // _CH) * _CH
_TAIL = _N - _FULL
_LAST0 = _TOT_CH - 14
_STAGE_A = _FAST_HI * _CH
_LIM_A = (_N - _STAGE_A) // _CH
_PEN0 = _LAST0 - 14
_LAST_W = (_N - _LAST0 * _CH)

_mesh = plsc.VectorSubcoreMesh(core_axis_name="c", subcore_axis_name="s")


@functools.partial(
    pl.kernel,
    mesh=_mesh,
    out_type=jax.ShapeDtypeStruct((_N, _D), jnp.float32),
    scratch_types=[
        pltpu.VMEM((_FAST_HI * _CH,), jnp.int32),
        pltpu.VMEM((_NBUF, _CH, _D), jnp.float32),
    ] + [pltpu.SemaphoreType.DMA] * _NBUF,
)
def _emb_gather(idx_hbm, emb_hbm, out_hbm, idx_v, rows_v, *sems):
    cid = lax.axis_index("c")
    sid = lax.axis_index("s")
    on_fast = cid == _FAST_CORE
    extra = jnp.maximum(sid - 7, 0)
    my_nch = jnp.where(on_fast, _FAST_HI - (sid >= 7), _SLOW_HI - (sid >= 7))
    chunk0 = jnp.where(on_fast, _FAST_HI * sid - extra,
                       _FAST_CH + _SLOW_HI * sid - extra)

    @pl.when(chunk0 <= _LIM_A)
    def _stage_main():
        pltpu.sync_copy(idx_hbm.at[pl.ds(chunk0 * _CH, _STAGE_A)], idx_v)

    @pl.when(chunk0 == _PEN0)
    def _stage_penultimate():
        pltpu.sync_copy(idx_hbm.at[pl.ds(_PEN0 * _CH, 14 * _CH)],
                        idx_v.at[pl.ds(0, 14 * _CH)])

    @pl.when(chunk0 == _LAST0)
    def _stage_last():
        pltpu.sync_copy(idx_hbm.at[pl.ds(_LAST0 * _CH, _LAST_W)],
                        idx_v.at[pl.ds(0, _LAST_W)])
        for t in range(_LAST_W, 14 * _CH, 16):
            idx_v[pl.ds(t, 16)] = jnp.zeros((16,), jnp.int32)

    def _issue(k):
        b = k % _NBUF

        @pl.when(k < my_nch)
        def _():
            pltpu.async_copy(emb_hbm.at[idx_v.at[pl.ds(k * _CH, _CH)]],
                             rows_v.at[b], sems[b])

    for k in range(_NBUF - 1):
        _issue(k)

    for j in range(_FAST_HI):
        b = j % _NBUF

        if j + _NBUF - 1 < _FAST_HI:
            _issue(j + _NBUF - 1)

        @pl.when(j < my_nch)
        def _wait_and_write(b=b, j=j):
            pltpu.make_async_copy(emb_hbm.at[idx_v.at[pl.ds(j * _CH, _CH)]],
                                  rows_v.at[b], sems[b]).wait()
            rbase = (chunk0 + j) * _CH

            @pl.when(rbase + _CH <= _N)
            def _full_write():
                pltpu.sync_copy(rows_v.at[b], out_hbm.at[pl.ds(rbase, _CH)])

            @pl.when(rbase == _FULL)
            def _tail_write():
                pltpu.sync_copy(rows_v.at[b].at[pl.ds(0, _TAIL)],
                                out_hbm.at[pl.ds(_FULL, _TAIL)])


def kernel(node_val, emb):
    return _emb_gather(node_val.reshape(-1), emb)
